# baseline (speedup 1.0000x reference)
import functools

import jax
import jax.numpy as jnp
from jax import lax
from jax.experimental import pallas as pl
from jax.experimental.pallas import tpu as pltpu
from jax.experimental.pallas import tpu_sc as plsc

N_NODES = 10000
N_PAD = 10240
N_EDGES = 320000
D = 128
DH = D // 2

CHUNK = 128
SL = CHUNK // 16
NROWS = N_EDGES // CHUNK
NFULL = NROWS // 16
NEXTRA = NROWS - 16 * NFULL
RING = 2
NBLK = (NFULL + 1) // RING
HALF = NFULL // 2
ZROWS = N_PAD // 16
WB = 624
WB_LAST = N_NODES - 15 * WB


def _sc_segment_sum(x2, src2d, dst2d):
    mesh = plsc.VectorSubcoreMesh(core_axis_name="c", subcore_axis_name="s")

    @functools.partial(
        pl.kernel,
        out_type=[
            jax.ShapeDtypeStruct((2, N_NODES, DH), jnp.float32),
            jax.ShapeDtypeStruct((2, N_NODES, 16), jnp.float32),
        ],
        mesh=mesh,
        compiler_params=pltpu.CompilerParams(use_tc_tiling_on_sc=False),
        scratch_types=[
            pltpu.VMEM((NFULL + 1 + RING, CHUNK), jnp.int32),
            pltpu.VMEM((NFULL + 1, CHUNK), jnp.int32),
            pltpu.VMEM((CHUNK, DH), jnp.float32),
            pltpu.VMEM((CHUNK, DH), jnp.float32),
            pltpu.VMEM((CHUNK, 16), jnp.float32),
            pltpu.VMEM((ZROWS // 2, 16), jnp.float32),
            pltpu.VMEM_SHARED((N_PAD, DH), jnp.float32),
            pltpu.VMEM_SHARED((N_PAD, 16), jnp.float32),
        ] + [pltpu.SemaphoreType.DMA] * (2 * RING + 1),
    )
    def k(x_hbm, src_hbm, dst_hbm, sum_out, deg_out,
          src_blk, dst_blk, rows0, rows1,
          ones_v, z16_v, acc_sh, deg_sh, *sems):
        rows = [rows0, rows1]
        gsem = sems[:RING]
        ssem = sems[RING:2 * RING]
        dsem = sems[2 * RING]
        c = lax.axis_index("c")
        s = lax.axis_index("s")
        has_extra = s < NEXTRA

        base = s * NFULL + jnp.minimum(s, NEXTRA)
        pltpu.sync_copy(src_hbm.at[pl.ds(base, NFULL)],
                        src_blk.at[pl.ds(0, NFULL)])
        pltpu.sync_copy(dst_hbm.at[pl.ds(base, NFULL)],
                        dst_blk.at[pl.ds(0, NFULL)])

        @pl.when(has_extra)
        def _():
            pltpu.sync_copy(src_hbm.at[pl.ds(base + NFULL, 1)],
                            src_blk.at[pl.ds(NFULL, 1)])
            pltpu.sync_copy(dst_hbm.at[pl.ds(base + NFULL, 1)],
                            dst_blk.at[pl.ds(NFULL, 1)])

        def _xform_row(j):
            for q in range(SL):
                sl = pl.ds(q * 16, 16)
                v = src_blk[j, sl]
                src_blk[j, sl] = v + v + c

        def pz(i, _):
            src_blk[NFULL + 1 + i // SL, pl.ds((i % SL) * 16, 16)] = (
                jnp.zeros((16,), jnp.int32))
            return 0
        lax.fori_loop(0, RING * SL, pz, 0)

        @pl.when(jnp.logical_not(has_extra))
        def _():
            def pz2(i, _):
                src_blk[NFULL, pl.ds(i * 16, 16)] = jnp.zeros((16,), jnp.int32)
                return 0
            lax.fori_loop(0, SL, pz2, 0)

        def xf(i, _):
            _xform_row(i)
            return 0
        lax.fori_loop(0, RING, xf, 0)

        def zf16(i, _):
            z16_v[i, :] = jnp.zeros((16,), jnp.float32)
            return 0
        lax.fori_loop(0, ZROWS // 2, zf16, 0)

        def of(i, _):
            ones_v[i, :] = jnp.ones((16,), jnp.float32)
            return 0
        lax.fori_loop(0, CHUNK, of, 0)

        def zf(i, _):
            rows0[i // (DH // 16), pl.ds((i % (DH // 16)) * 16, 16)] = (
                jnp.zeros((16,), jnp.float32))
            return 0
        lax.fori_loop(0, CHUNK * (DH // 16), zf, 0)
        zbase = s * ZROWS
        for r in range(ZROWS // CHUNK):
            pltpu.sync_copy(rows0, acc_sh.at[pl.ds(zbase + r * CHUNK, CHUNK)])
        pltpu.sync_copy(z16_v, deg_sh.at[pl.ds(zbase, ZROWS // 2)])
        pltpu.sync_copy(z16_v, deg_sh.at[pl.ds(zbase + ZROWS // 2, ZROWS // 2)])

        for b in range(RING):
            pltpu.async_copy(x_hbm.at[src_blk.at[b]], rows[b], gsem[b])
        plsc.subcore_barrier()

        def step(g, _):
            base_e = g * RING
            for b in range(RING):
                e = base_e + b
                my_deg = (e < HALF) == (c == 0)
                pltpu.make_async_copy(x_hbm.at[src_blk.at[e]],
                                      rows[b], gsem[b]).wait()
                pltpu.async_copy(rows[b], acc_sh.at[dst_blk.at[e]],
                                 ssem[b], add=True)

                @pl.when(my_deg)
                def _():
                    pltpu.async_copy(ones_v, deg_sh.at[dst_blk.at[e]],
                                     dsem, add=True)

            for b in range(RING):
                e = base_e + b
                _xform_row(e + RING)
                pltpu.make_async_copy(rows[b], acc_sh.at[dst_blk.at[e]],
                                      ssem[b]).wait()
                pltpu.async_copy(x_hbm.at[src_blk.at[e + RING]],
                                 rows[b], gsem[b])

            for b in range(RING):
                e = base_e + b
                my_deg = (e < HALF) == (c == 0)

                @pl.when(my_deg)
                def _():
                    pltpu.make_async_copy(ones_v, deg_sh.at[dst_blk.at[e]],
                                          dsem).wait()
            return 0
        lax.fori_loop(0, NBLK, step, 0)

        pltpu.make_async_copy(x_hbm.at[src_blk.at[NFULL]], rows[0],
                              gsem[0]).wait()
        pltpu.make_async_copy(x_hbm.at[src_blk.at[NFULL + 1]], rows[1],
                              gsem[1]).wait()

        @pl.when(has_extra)
        def _():
            pltpu.async_copy(rows[0], acc_sh.at[dst_blk.at[NFULL]],
                             ssem[0], add=True)

            @pl.when(c == 1)
            def _():
                pltpu.async_copy(ones_v, deg_sh.at[dst_blk.at[NFULL]],
                                 dsem, add=True)
                pltpu.make_async_copy(ones_v, deg_sh.at[dst_blk.at[NFULL]],
                                      dsem).wait()
            pltpu.make_async_copy(rows[0], acc_sh.at[dst_blk.at[NFULL]],
                                  ssem[0]).wait()
        plsc.subcore_barrier()

        rbase = s * WB

        @pl.when(s < 15)
        def _():
            pltpu.sync_copy(acc_sh.at[pl.ds(rbase, WB)],
                            sum_out.at[c, pl.ds(rbase, WB)])
            pltpu.sync_copy(deg_sh.at[pl.ds(rbase, WB)],
                            deg_out.at[c, pl.ds(rbase, WB)])

        @pl.when(s == 15)
        def _():
            pltpu.sync_copy(acc_sh.at[pl.ds(rbase, WB_LAST)],
                            sum_out.at[c, pl.ds(rbase, WB_LAST)])
            pltpu.sync_copy(deg_sh.at[pl.ds(rbase, WB_LAST)],
                            deg_out.at[c, pl.ds(rbase, WB_LAST)])

    return k(x2, src2d, dst2d)


def _tc_combine(x, sL, sR, dp0, dp1, W_self, W_neigh, b):
    BLK = 1000
    grid = (N_NODES // BLK,)

    def body(x_ref, sl_ref, sr_ref, d0_ref, d1_ref, ws_ref, wn_ref, b_ref,
             o_ref):
        deg = d0_ref[:, 0:1] + d1_ref[:, 0:1]
        inv = 1.0 / jnp.maximum(deg, 1.0)
        h = jnp.concatenate([sl_ref[...], sr_ref[...]], axis=1) * inv
        o_ref[...] = (
            jnp.dot(x_ref[...], ws_ref[...], preferred_element_type=jnp.float32)
            + jnp.dot(h, wn_ref[...], preferred_element_type=jnp.float32)
            + b_ref[...]
        )

    return pl.pallas_call(
        body,
        grid=grid,
        in_specs=[
            pl.BlockSpec((BLK, D), lambda i: (i, 0)),
            pl.BlockSpec((BLK, DH), lambda i: (i, 0)),
            pl.BlockSpec((BLK, DH), lambda i: (i, 0)),
            pl.BlockSpec((BLK, 16), lambda i: (i, 0)),
            pl.BlockSpec((BLK, 16), lambda i: (i, 0)),
            pl.BlockSpec((D, D), lambda i: (0, 0)),
            pl.BlockSpec((D, D), lambda i: (0, 0)),
            pl.BlockSpec((1, D), lambda i: (0, 0)),
        ],
        out_specs=pl.BlockSpec((BLK, D), lambda i: (i, 0)),
        out_shape=jax.ShapeDtypeStruct((N_NODES, D), jnp.float32),
    )(x, sL, sR, dp0, dp1, W_self, W_neigh, b)


def kernel(x, edge_index, W_self, W_neigh, b):
    src2d = edge_index[0].astype(jnp.int32).reshape(NROWS, CHUNK)
    dst2d = edge_index[1].astype(jnp.int32).reshape(NROWS, CHUNK)
    x2 = x.reshape(2 * N_NODES, DH)
    sums, deg = _sc_segment_sum(x2, src2d, dst2d)
    return _tc_combine(x, sums[0], sums[1], deg[0], deg[1],
                       W_self, W_neigh, b.reshape(1, D))


# speedup vs baseline: 1.0292x; 1.0292x over previous
import functools

import jax
import jax.numpy as jnp
from jax import lax
from jax.experimental import pallas as pl
from jax.experimental.pallas import tpu as pltpu
from jax.experimental.pallas import tpu_sc as plsc

N_NODES = 10000
N_PAD = 10240
N_EDGES = 320000
D = 128
DH = D // 2

EDGES_PER_TILE = N_EDGES // 16
CHUNK = 80
NCHUNK = EDGES_PER_TILE // CHUNK
HALF = NCHUNK // 2
ZROWS = N_PAD // 16
WB = 624
WB_LAST = N_NODES - 15 * WB


def _sc_segment_sum(x2, src, dst):
    mesh = plsc.VectorSubcoreMesh(core_axis_name="c", subcore_axis_name="s")

    @functools.partial(
        pl.kernel,
        out_type=[
            jax.ShapeDtypeStruct((2, N_NODES, DH), jnp.float32),
            jax.ShapeDtypeStruct((2, N_NODES, 16), jnp.float32),
        ],
        mesh=mesh,
        compiler_params=pltpu.CompilerParams(use_tc_tiling_on_sc=False),
        scratch_types=[
            pltpu.VMEM((NCHUNK + 2, CHUNK), jnp.int32),
            pltpu.VMEM((NCHUNK, CHUNK), jnp.int32),
            pltpu.VMEM((CHUNK, DH), jnp.float32),
            pltpu.VMEM((CHUNK, DH), jnp.float32),
            pltpu.VMEM((CHUNK, 16), jnp.float32),
            pltpu.VMEM((ZROWS, 16), jnp.float32),
            pltpu.VMEM_SHARED((N_PAD, DH), jnp.float32),
            pltpu.VMEM_SHARED((N_PAD, 16), jnp.float32),
            pltpu.SemaphoreType.DMA,
            pltpu.SemaphoreType.DMA,
            pltpu.SemaphoreType.DMA,
            pltpu.SemaphoreType.DMA,
            pltpu.SemaphoreType.DMA,
        ],
    )
    def k(x_hbm, src_hbm, dst_hbm, sum_out, deg_out,
          src_blk, dst_blk, rows0, rows1, ones_v, z16_v, acc_sh, deg_sh,
          gsem0, gsem1, ssem0, ssem1, dsem):
        c = lax.axis_index("c")
        s = lax.axis_index("s")

        pltpu.sync_copy(src_hbm.at[pl.ds(s * NCHUNK, NCHUNK)],
                        src_blk.at[pl.ds(0, NCHUNK)])
        pltpu.sync_copy(dst_hbm.at[pl.ds(s * NCHUNK, NCHUNK)], dst_blk)

        SL = CHUNK // 16

        def xf(i, _):
            j = i // SL
            sl = pl.ds((i % SL) * 16, 16)
            v = src_blk[j, sl]
            src_blk[j, sl] = v + v + c
            return 0
        lax.fori_loop(0, NCHUNK * SL, xf, 0)

        def pf(i, _):
            src_blk[NCHUNK + i // SL, pl.ds((i % SL) * 16, 16)] = (
                jnp.zeros((16,), jnp.int32))
            return 0
        lax.fori_loop(0, 2 * SL, pf, 0)

        def zf(i, _):
            rows0[i // (DH // 16), pl.ds((i % (DH // 16)) * 16, 16)] = (
                jnp.zeros((16,), jnp.float32))
            return 0
        lax.fori_loop(0, CHUNK * (DH // 16), zf, 0)

        def zf16(i, _):
            z16_v[i, :] = jnp.zeros((16,), jnp.float32)
            return 0
        lax.fori_loop(0, ZROWS, zf16, 0)

        def of(i, _):
            ones_v[i, :] = jnp.ones((16,), jnp.float32)
            return 0
        lax.fori_loop(0, CHUNK, of, 0)

        zbase = s * ZROWS
        for r in range(ZROWS // CHUNK):
            pltpu.sync_copy(rows0, acc_sh.at[pl.ds(zbase + r * CHUNK, CHUNK)])
        pltpu.sync_copy(z16_v, deg_sh.at[pl.ds(zbase, ZROWS)])

        pltpu.async_copy(x_hbm.at[src_blk.at[0]], rows0, gsem0)
        pltpu.async_copy(x_hbm.at[src_blk.at[1]], rows1, gsem1)
        plsc.subcore_barrier()

        def step(g, _):
            e0 = 2 * g
            e1 = e0 + 1
            pltpu.make_async_copy(x_hbm.at[src_blk.at[e0]], rows0, gsem0).wait()
            pltpu.async_copy(rows0, acc_sh.at[dst_blk.at[e0]], ssem0, add=True)

            @pl.when(c == 0)
            def _():
                pltpu.async_copy(ones_v, deg_sh.at[dst_blk.at[e0]], dsem,
                                 add=True)

            pltpu.make_async_copy(x_hbm.at[src_blk.at[e1]], rows1, gsem1).wait()
            pltpu.async_copy(rows1, acc_sh.at[dst_blk.at[e1]], ssem1, add=True)

            @pl.when(c == 0)
            def _():
                pltpu.async_copy(ones_v, deg_sh.at[dst_blk.at[e1]], dsem,
                                 add=True)

            pltpu.make_async_copy(rows0, acc_sh.at[dst_blk.at[e0]], ssem0).wait()
            pltpu.async_copy(x_hbm.at[src_blk.at[e0 + 2]], rows0, gsem0)
            pltpu.make_async_copy(rows1, acc_sh.at[dst_blk.at[e1]], ssem1).wait()
            pltpu.async_copy(x_hbm.at[src_blk.at[e1 + 2]], rows1, gsem1)

            @pl.when(c == 0)
            def _():
                pltpu.make_async_copy(ones_v, deg_sh.at[dst_blk.at[e0]],
                                      dsem).wait()
                pltpu.make_async_copy(ones_v, deg_sh.at[dst_blk.at[e1]],
                                      dsem).wait()
            return 0
        lax.fori_loop(0, NCHUNK // 2, step, 0)

        pltpu.make_async_copy(x_hbm.at[src_blk.at[NCHUNK]], rows0, gsem0).wait()
        pltpu.make_async_copy(x_hbm.at[src_blk.at[NCHUNK + 1]], rows1,
                              gsem1).wait()
        plsc.subcore_barrier()

        rbase = s * WB

        @pl.when(s < 15)
        def _():
            pltpu.sync_copy(acc_sh.at[pl.ds(rbase, WB)],
                            sum_out.at[c, pl.ds(rbase, WB)])
            pltpu.sync_copy(deg_sh.at[pl.ds(rbase, WB)],
                            deg_out.at[c, pl.ds(rbase, WB)])

        @pl.when(s == 15)
        def _():
            pltpu.sync_copy(acc_sh.at[pl.ds(rbase, WB_LAST)],
                            sum_out.at[c, pl.ds(rbase, WB_LAST)])
            pltpu.sync_copy(deg_sh.at[pl.ds(rbase, WB_LAST)],
                            deg_out.at[c, pl.ds(rbase, WB_LAST)])

    return k(x2, src, dst)


def _tc_combine(x, sL, sR, dp0, dp1, W_self, W_neigh, b):
    BLK = 1000
    grid = (N_NODES // BLK,)

    def body(x_ref, sl_ref, sr_ref, d0_ref, d1_ref, ws_ref, wn_ref, b_ref,
             o_ref):
        deg = d0_ref[:, 0:1] + d1_ref[:, 0:1]
        inv = 1.0 / jnp.maximum(deg, 1.0)
        h = jnp.concatenate([sl_ref[...], sr_ref[...]], axis=1) * inv
        o_ref[...] = (
            jnp.dot(x_ref[...], ws_ref[...], preferred_element_type=jnp.float32)
            + jnp.dot(h, wn_ref[...], preferred_element_type=jnp.float32)
            + b_ref[...]
        )

    return pl.pallas_call(
        body,
        grid=grid,
        in_specs=[
            pl.BlockSpec((BLK, D), lambda i: (i, 0)),
            pl.BlockSpec((BLK, DH), lambda i: (i, 0)),
            pl.BlockSpec((BLK, DH), lambda i: (i, 0)),
            pl.BlockSpec((BLK, 16), lambda i: (i, 0)),
            pl.BlockSpec((BLK, 16), lambda i: (i, 0)),
            pl.BlockSpec((D, D), lambda i: (0, 0)),
            pl.BlockSpec((D, D), lambda i: (0, 0)),
            pl.BlockSpec((1, D), lambda i: (0, 0)),
        ],
        out_specs=pl.BlockSpec((BLK, D), lambda i: (i, 0)),
        out_shape=jax.ShapeDtypeStruct((N_NODES, D), jnp.float32),
    )(x, sL, sR, dp0, dp1, W_self, W_neigh, b)


def kernel(x, edge_index, W_self, W_neigh, b):
    src2d = edge_index[0].astype(jnp.int32).reshape(N_EDGES // CHUNK, CHUNK)
    dst2d = edge_index[1].astype(jnp.int32).reshape(N_EDGES // CHUNK, CHUNK)
    x2 = x.reshape(2 * N_NODES, DH)
    sums, deg = _sc_segment_sum(x2, src2d, dst2d)
    return _tc_combine(x, sums[0], sums[1], deg[0], deg[1],
                       W_self, W_neigh, b.reshape(1, D))


# speedup vs baseline: 1.0666x; 1.0363x over previous
import functools

import jax
import jax.numpy as jnp
from jax import lax
from jax.experimental import pallas as pl
from jax.experimental.pallas import tpu as pltpu
from jax.experimental.pallas import tpu_sc as plsc

N_NODES = 10000
N_PAD = 10240
N_EDGES = 320000
D = 128
DH = D // 2

EDGES_PER_TILE = N_EDGES // 16
CHUNK = 80
NCHUNK = EDGES_PER_TILE // CHUNK
HALF = NCHUNK // 2
ZROWS = N_PAD // 16
WB = 624
WB_LAST = N_NODES - 15 * WB


def _sc_segment_sum(x2, src, dst):
    mesh = plsc.VectorSubcoreMesh(core_axis_name="c", subcore_axis_name="s")

    @functools.partial(
        pl.kernel,
        out_type=[
            jax.ShapeDtypeStruct((2, N_NODES, DH), jnp.float32),
            jax.ShapeDtypeStruct((N_NODES, 16), jnp.float32),
        ],
        mesh=mesh,
        compiler_params=pltpu.CompilerParams(use_tc_tiling_on_sc=False),
        scratch_types=[
            pltpu.VMEM((NCHUNK + 2, CHUNK), jnp.int32),
            pltpu.VMEM((NCHUNK, CHUNK), jnp.int32),
            pltpu.VMEM((CHUNK, DH), jnp.float32),
            pltpu.VMEM((CHUNK, DH), jnp.float32),
            pltpu.VMEM((CHUNK, 16), jnp.float32),
            pltpu.VMEM((ZROWS, 16), jnp.float32),
            pltpu.VMEM_SHARED((N_PAD, DH), jnp.float32),
            pltpu.VMEM_SHARED((N_PAD, 16), jnp.float32),
            pltpu.SemaphoreType.DMA,
            pltpu.SemaphoreType.DMA,
            pltpu.SemaphoreType.DMA,
            pltpu.SemaphoreType.DMA,
            pltpu.SemaphoreType.DMA,
        ],
    )
    def k(x_hbm, src_hbm, dst_hbm, sum_out, deg_out,
          src_blk, dst_blk, rows0, rows1, ones_v, z16_v, acc_sh, deg_sh,
          gsem0, gsem1, ssem0, ssem1, dsem):
        c = lax.axis_index("c")
        s = lax.axis_index("s")

        pltpu.sync_copy(src_hbm.at[pl.ds(s * NCHUNK, NCHUNK)],
                        src_blk.at[pl.ds(0, NCHUNK)])
        pltpu.sync_copy(dst_hbm.at[pl.ds(s * NCHUNK, NCHUNK)], dst_blk)

        SL = CHUNK // 16

        def xf(i, _):
            j = i // SL
            sl = pl.ds((i % SL) * 16, 16)
            v = src_blk[j, sl]
            src_blk[j, sl] = v + v + c
            return 0
        lax.fori_loop(0, NCHUNK * SL, xf, 0)

        def pf(i, _):
            src_blk[NCHUNK + i // SL, pl.ds((i % SL) * 16, 16)] = (
                jnp.zeros((16,), jnp.int32))
            return 0
        lax.fori_loop(0, 2 * SL, pf, 0)

        def zf(i, _):
            rows0[i // (DH // 16), pl.ds((i % (DH // 16)) * 16, 16)] = (
                jnp.zeros((16,), jnp.float32))
            return 0
        lax.fori_loop(0, CHUNK * (DH // 16), zf, 0)

        def zf16(i, _):
            z16_v[i, :] = jnp.zeros((16,), jnp.float32)
            return 0
        lax.fori_loop(0, ZROWS, zf16, 0)

        def of(i, _):
            ones_v[i, :] = jnp.ones((16,), jnp.float32)
            return 0
        lax.fori_loop(0, CHUNK, of, 0)

        zbase = s * ZROWS
        for r in range(ZROWS // CHUNK):
            pltpu.sync_copy(rows0, acc_sh.at[pl.ds(zbase + r * CHUNK, CHUNK)])
        pltpu.sync_copy(z16_v, deg_sh.at[pl.ds(zbase, ZROWS)])

        pltpu.async_copy(x_hbm.at[src_blk.at[0]], rows0, gsem0)
        pltpu.async_copy(x_hbm.at[src_blk.at[1]], rows1, gsem1)
        plsc.subcore_barrier()

        def step(g, _):
            e0 = 2 * g
            e1 = e0 + 1
            pltpu.make_async_copy(x_hbm.at[src_blk.at[e0]], rows0, gsem0).wait()
            pltpu.async_copy(rows0, acc_sh.at[dst_blk.at[e0]], ssem0, add=True)

            @pl.when(c == 0)
            def _():
                pltpu.async_copy(ones_v, deg_sh.at[dst_blk.at[e0]], dsem,
                                 add=True)

            pltpu.make_async_copy(x_hbm.at[src_blk.at[e1]], rows1, gsem1).wait()
            pltpu.async_copy(rows1, acc_sh.at[dst_blk.at[e1]], ssem1, add=True)

            @pl.when(c == 0)
            def _():
                pltpu.async_copy(ones_v, deg_sh.at[dst_blk.at[e1]], dsem,
                                 add=True)

            pltpu.make_async_copy(rows0, acc_sh.at[dst_blk.at[e0]], ssem0).wait()
            pltpu.async_copy(x_hbm.at[src_blk.at[e0 + 2]], rows0, gsem0)
            pltpu.make_async_copy(rows1, acc_sh.at[dst_blk.at[e1]], ssem1).wait()
            pltpu.async_copy(x_hbm.at[src_blk.at[e1 + 2]], rows1, gsem1)

            @pl.when(c == 0)
            def _():
                pltpu.make_async_copy(ones_v, deg_sh.at[dst_blk.at[e0]],
                                      dsem).wait()
                pltpu.make_async_copy(ones_v, deg_sh.at[dst_blk.at[e1]],
                                      dsem).wait()
            return 0
        lax.fori_loop(0, NCHUNK // 2, step, 0)

        pltpu.make_async_copy(x_hbm.at[src_blk.at[NCHUNK]], rows0, gsem0).wait()
        pltpu.make_async_copy(x_hbm.at[src_blk.at[NCHUNK + 1]], rows1,
                              gsem1).wait()
        plsc.subcore_barrier()

        rbase = s * WB

        @pl.when(s < 15)
        def _():
            pltpu.sync_copy(acc_sh.at[pl.ds(rbase, WB)],
                            sum_out.at[c, pl.ds(rbase, WB)])

            @pl.when(c == 0)
            def _():
                pltpu.sync_copy(deg_sh.at[pl.ds(rbase, WB)],
                                deg_out.at[pl.ds(rbase, WB)])

        @pl.when(s == 15)
        def _():
            pltpu.sync_copy(acc_sh.at[pl.ds(rbase, WB_LAST)],
                            sum_out.at[c, pl.ds(rbase, WB_LAST)])

            @pl.when(c == 0)
            def _():
                pltpu.sync_copy(deg_sh.at[pl.ds(rbase, WB_LAST)],
                                deg_out.at[pl.ds(rbase, WB_LAST)])

    return k(x2, src, dst)


def _tc_combine(x, sL, sR, dp, W_self, W_neigh, b):
    BLK = 1000
    grid = (N_NODES // BLK,)

    def body(x_ref, sl_ref, sr_ref, d_ref, ws_ref, wn_ref, b_ref, o_ref):
        inv = 1.0 / jnp.maximum(d_ref[:, 0:1], 1.0)
        h = jnp.concatenate([sl_ref[...], sr_ref[...]], axis=1) * inv
        o_ref[...] = (
            jnp.dot(x_ref[...], ws_ref[...], preferred_element_type=jnp.float32)
            + jnp.dot(h, wn_ref[...], preferred_element_type=jnp.float32)
            + b_ref[...]
        )

    return pl.pallas_call(
        body,
        grid=grid,
        in_specs=[
            pl.BlockSpec((BLK, D), lambda i: (i, 0)),
            pl.BlockSpec((BLK, DH), lambda i: (i, 0)),
            pl.BlockSpec((BLK, DH), lambda i: (i, 0)),
            pl.BlockSpec((BLK, 16), lambda i: (i, 0)),
            pl.BlockSpec((D, D), lambda i: (0, 0)),
            pl.BlockSpec((D, D), lambda i: (0, 0)),
            pl.BlockSpec((1, D), lambda i: (0, 0)),
        ],
        out_specs=pl.BlockSpec((BLK, D), lambda i: (i, 0)),
        out_shape=jax.ShapeDtypeStruct((N_NODES, D), jnp.float32),
    )(x, sL, sR, dp, W_self, W_neigh, b)


def kernel(x, edge_index, W_self, W_neigh, b):
    src2d = edge_index[0].astype(jnp.int32).reshape(N_EDGES // CHUNK, CHUNK)
    dst2d = edge_index[1].astype(jnp.int32).reshape(N_EDGES // CHUNK, CHUNK)
    x2 = x.reshape(2 * N_NODES, DH)
    sums, deg = _sc_segment_sum(x2, src2d, dst2d)
    return _tc_combine(x, sums[0], sums[1], deg,
                       W_self, W_neigh, b.reshape(1, D))


# speedup vs baseline: 1.0876x; 1.0197x over previous
import functools

import jax
import jax.numpy as jnp
from jax import lax
from jax.experimental import pallas as pl
from jax.experimental.pallas import tpu as pltpu
from jax.experimental.pallas import tpu_sc as plsc

N_NODES = 10000
N_PAD = 10240
N_EDGES = 320000
D = 128
DH = D // 2

EDGES_PER_TILE = N_EDGES // 16
CHUNK = 80
NCHUNK = EDGES_PER_TILE // CHUNK
HALF = NCHUNK // 2
ZROWS = N_PAD // 16
WB = 624
WB_LAST = N_NODES - 15 * WB


def _sc_segment_sum(x2, src, dst):
    mesh = plsc.VectorSubcoreMesh(core_axis_name="c", subcore_axis_name="s")

    @functools.partial(
        pl.kernel,
        out_type=[
            jax.ShapeDtypeStruct((2, N_NODES, DH), jnp.float32),
            jax.ShapeDtypeStruct((N_NODES, 16), jnp.float32),
        ],
        mesh=mesh,
        compiler_params=pltpu.CompilerParams(use_tc_tiling_on_sc=False),
        scratch_types=[
            pltpu.VMEM((NCHUNK + 2, CHUNK), jnp.int32),
            pltpu.VMEM((NCHUNK, CHUNK), jnp.int32),
            pltpu.VMEM((CHUNK, DH), jnp.float32),
            pltpu.VMEM((CHUNK, DH), jnp.float32),
            pltpu.VMEM((CHUNK, 16), jnp.float32),
            pltpu.VMEM((ZROWS // 2, 16), jnp.float32),
            pltpu.VMEM_SHARED((N_PAD, DH), jnp.float32),
            pltpu.VMEM_SHARED((N_PAD, 16), jnp.float32),
            pltpu.SemaphoreType.DMA,
            pltpu.SemaphoreType.DMA,
            pltpu.SemaphoreType.DMA,
            pltpu.SemaphoreType.DMA,
            pltpu.SemaphoreType.DMA,
        ],
    )
    def k(x_hbm, src_hbm, dst_hbm, sum_out, deg_out,
          src_blk, dst_blk, rows0, rows1, ones_v, z16_v, acc_sh, deg_sh,
          gsem0, gsem1, ssem0, ssem1, dsem):
        c = lax.axis_index("c")
        s = lax.axis_index("s")

        pltpu.sync_copy(src_hbm.at[pl.ds(s * NCHUNK, NCHUNK)],
                        src_blk.at[pl.ds(0, NCHUNK)])
        pltpu.sync_copy(dst_hbm.at[pl.ds(s * NCHUNK, NCHUNK)], dst_blk)

        SL = CHUNK // 16

        def _xform_row(j):
            for q in range(SL):
                sl = pl.ds(q * 16, 16)
                v = src_blk[j, sl]
                src_blk[j, sl] = v + v + c

        def xf(i, _):
            _xform_row(i)
            return 0
        lax.fori_loop(0, 2, xf, 0)

        def pf(i, _):
            src_blk[NCHUNK + i // SL, pl.ds((i % SL) * 16, 16)] = (
                jnp.zeros((16,), jnp.int32))
            return 0
        lax.fori_loop(0, 2 * SL, pf, 0)

        def zf(i, _):
            rows0[i // (DH // 16), pl.ds((i % (DH // 16)) * 16, 16)] = (
                jnp.zeros((16,), jnp.float32))
            return 0
        lax.fori_loop(0, CHUNK * (DH // 16), zf, 0)

        def zf16(i, _):
            z16_v[i, :] = jnp.zeros((16,), jnp.float32)
            return 0
        lax.fori_loop(0, ZROWS // 2, zf16, 0)

        def of(i, _):
            ones_v[i, :] = jnp.ones((16,), jnp.float32)
            return 0
        lax.fori_loop(0, CHUNK, of, 0)

        zbase = s * ZROWS
        for r in range(ZROWS // CHUNK):
            pltpu.sync_copy(rows0, acc_sh.at[pl.ds(zbase + r * CHUNK, CHUNK)])
        pltpu.sync_copy(z16_v, deg_sh.at[pl.ds(zbase, ZROWS // 2)])
        pltpu.sync_copy(z16_v, deg_sh.at[pl.ds(zbase + ZROWS // 2, ZROWS // 2)])

        pltpu.async_copy(x_hbm.at[src_blk.at[0]], rows0, gsem0)
        pltpu.async_copy(x_hbm.at[src_blk.at[1]], rows1, gsem1)
        plsc.subcore_barrier()

        def step(g, _):
            e0 = 2 * g
            e1 = e0 + 1
            pltpu.make_async_copy(x_hbm.at[src_blk.at[e0]], rows0, gsem0).wait()
            pltpu.async_copy(rows0, acc_sh.at[dst_blk.at[e0]], ssem0, add=True)

            @pl.when(c == 0)
            def _():
                pltpu.async_copy(ones_v, deg_sh.at[dst_blk.at[e0]], dsem,
                                 add=True)

            pltpu.make_async_copy(x_hbm.at[src_blk.at[e1]], rows1, gsem1).wait()
            pltpu.async_copy(rows1, acc_sh.at[dst_blk.at[e1]], ssem1, add=True)

            @pl.when(c == 0)
            def _():
                pltpu.async_copy(ones_v, deg_sh.at[dst_blk.at[e1]], dsem,
                                 add=True)

            _xform_row(e0 + 2)
            pltpu.make_async_copy(rows0, acc_sh.at[dst_blk.at[e0]], ssem0).wait()
            pltpu.async_copy(x_hbm.at[src_blk.at[e0 + 2]], rows0, gsem0)
            _xform_row(e1 + 2)
            pltpu.make_async_copy(rows1, acc_sh.at[dst_blk.at[e1]], ssem1).wait()
            pltpu.async_copy(x_hbm.at[src_blk.at[e1 + 2]], rows1, gsem1)

            @pl.when(c == 0)
            def _():
                pltpu.make_async_copy(ones_v, deg_sh.at[dst_blk.at[e0]],
                                      dsem).wait()
                pltpu.make_async_copy(ones_v, deg_sh.at[dst_blk.at[e1]],
                                      dsem).wait()
            return 0
        lax.fori_loop(0, NCHUNK // 2, step, 0)

        pltpu.make_async_copy(x_hbm.at[src_blk.at[NCHUNK]], rows0, gsem0).wait()
        pltpu.make_async_copy(x_hbm.at[src_blk.at[NCHUNK + 1]], rows1,
                              gsem1).wait()
        plsc.subcore_barrier()

        rbase = s * WB

        @pl.when(s < 15)
        def _():
            pltpu.sync_copy(acc_sh.at[pl.ds(rbase, WB)],
                            sum_out.at[c, pl.ds(rbase, WB)])

            @pl.when(c == 0)
            def _():
                pltpu.sync_copy(deg_sh.at[pl.ds(rbase, WB)],
                                deg_out.at[pl.ds(rbase, WB)])

        @pl.when(s == 15)
        def _():
            pltpu.sync_copy(acc_sh.at[pl.ds(rbase, WB_LAST)],
                            sum_out.at[c, pl.ds(rbase, WB_LAST)])

            @pl.when(c == 0)
            def _():
                pltpu.sync_copy(deg_sh.at[pl.ds(rbase, WB_LAST)],
                                deg_out.at[pl.ds(rbase, WB_LAST)])

    return k(x2, src, dst)


def _tc_combine(x, sL, sR, dp, W_self, W_neigh, b):
    BLK = 1000
    grid = (N_NODES // BLK,)

    def body(x_ref, sl_ref, sr_ref, d_ref, ws_ref, wn_ref, b_ref, o_ref):
        inv = 1.0 / jnp.maximum(d_ref[:, 0:1], 1.0)
        h = jnp.concatenate([sl_ref[...], sr_ref[...]], axis=1) * inv
        o_ref[...] = (
            jnp.dot(x_ref[...], ws_ref[...], preferred_element_type=jnp.float32)
            + jnp.dot(h, wn_ref[...], preferred_element_type=jnp.float32)
            + b_ref[...]
        )

    return pl.pallas_call(
        body,
        grid=grid,
        in_specs=[
            pl.BlockSpec((BLK, D), lambda i: (i, 0)),
            pl.BlockSpec((BLK, DH), lambda i: (i, 0)),
            pl.BlockSpec((BLK, DH), lambda i: (i, 0)),
            pl.BlockSpec((BLK, 16), lambda i: (i, 0)),
            pl.BlockSpec((D, D), lambda i: (0, 0)),
            pl.BlockSpec((D, D), lambda i: (0, 0)),
            pl.BlockSpec((1, D), lambda i: (0, 0)),
        ],
        out_specs=pl.BlockSpec((BLK, D), lambda i: (i, 0)),
        out_shape=jax.ShapeDtypeStruct((N_NODES, D), jnp.float32),
    )(x, sL, sR, dp, W_self, W_neigh, b)


def kernel(x, edge_index, W_self, W_neigh, b):
    src2d = edge_index[0].astype(jnp.int32).reshape(N_EDGES // CHUNK, CHUNK)
    dst2d = edge_index[1].astype(jnp.int32).reshape(N_EDGES // CHUNK, CHUNK)
    x2 = x.reshape(2 * N_NODES, DH)
    sums, deg = _sc_segment_sum(x2, src2d, dst2d)
    return _tc_combine(x, sums[0], sums[1], deg,
                       W_self, W_neigh, b.reshape(1, D))
